# baseline (speedup 1.0000x reference)
import jax
import jax.numpy as jnp
from jax.experimental import pallas as pl
from jax.experimental.pallas import tpu as pltpu

_EPS = 1e-5


def _toeplitz(w_oihw, cin, w, kw, wout, cout):
    wk = (jnp.transpose(w_oihw[:, :, 0, :], (2, 1, 0))
          .reshape(kw * cin, cout))
    stride = w * cin
    ext = stride + cin
    col = jnp.concatenate(
        [wk, jnp.zeros((ext - kw * cin, cout), wk.dtype)], axis=0)
    slab = jnp.tile(col[None], (wout, 1, 1)).reshape(wout * ext, cout)
    w3 = slab[:wout * stride].reshape(wout, stride, cout)
    return (jnp.transpose(w3, (1, 0, 2))
            .reshape(stride, wout * cout).astype(jnp.bfloat16))


def _conv_stats_kernel(x_ref, w_ref, y_ref, sum_ref, ssq_ref):
    @pl.when(pl.program_id(1) == 0)
    def _():
        sum_ref[...] = jnp.zeros_like(sum_ref)
        ssq_ref[...] = jnp.zeros_like(ssq_ref)

    xb = x_ref[...].astype(jnp.bfloat16)
    y = jnp.dot(xb, w_ref[...], preferred_element_type=jnp.float32)
    y_ref[...] = y.astype(jnp.bfloat16)
    sum_ref[0] += jnp.sum(y, axis=0, keepdims=True)
    ssq_ref[0] += jnp.sum(y * y, axis=0, keepdims=True)


def _bn_relu_kernel(y_ref, scale_ref, shift_ref, o_ref):
    y = y_ref[...].astype(jnp.float32)
    o_ref[...] = jnp.maximum(y * scale_ref[...] + shift_ref[...], 0.0)


def kernel(x_nchw, w_oihw, bias, gamma, beta):
    del bias
    n, cin, h, w = x_nchw.shape
    cout, cin_w, kh, kw = w_oihw.shape
    assert kh == 1 and cin_w == cin and w >= kw
    wout = w - kw + 1
    m = n * h
    wc_in = w * cin
    wc_out = wout * cout

    x2d = jnp.transpose(x_nchw, (0, 2, 3, 1)).reshape(m, wc_in)
    w_toe = _toeplitz(w_oihw, cin, w, kw, wout, cout)

    tm = min(1024, m)
    tm = max(8, (tm // 8) * 8)
    m_pad = pl.cdiv(m, tm) * tm
    if m_pad != m:
        x2d = jnp.pad(x2d, ((0, m_pad - m), (0, 0)))
    n_tiles = m_pad // tm
    if n_tiles % 2 == 0:
        cores, tiles_per_core = 2, n_tiles // 2
    else:
        cores, tiles_per_core = 1, n_tiles

    y2d, lane_sum, lane_ssq = pl.pallas_call(
        _conv_stats_kernel,
        out_shape=(jax.ShapeDtypeStruct((m_pad, wc_out), jnp.bfloat16),
                   jax.ShapeDtypeStruct((cores, 1, wc_out), jnp.float32),
                   jax.ShapeDtypeStruct((cores, 1, wc_out), jnp.float32)),
        grid=(cores, tiles_per_core),
        in_specs=[pl.BlockSpec((tm, wc_in), lambda c, i, t=tiles_per_core: (c * t + i, 0)),
                  pl.BlockSpec((wc_in, wc_out), lambda c, i: (0, 0))],
        out_specs=(pl.BlockSpec((tm, wc_out), lambda c, i, t=tiles_per_core: (c * t + i, 0)),
                   pl.BlockSpec((1, 1, wc_out), lambda c, i: (c, 0, 0)),
                   pl.BlockSpec((1, 1, wc_out), lambda c, i: (c, 0, 0))),
        compiler_params=pltpu.CompilerParams(
            dimension_semantics=("parallel", "arbitrary")),
        cost_estimate=pl.CostEstimate(
            flops=2 * m_pad * wc_in * wc_out, transcendentals=0,
            bytes_accessed=4 * m_pad * wc_in + 2 * m_pad * wc_out
            + 2 * wc_in * wc_out),
    )(x2d, w_toe)

    cnt = float(m * wout)
    s = jnp.sum(lane_sum.reshape(cores, wout, cout), axis=(0, 1))
    sq = jnp.sum(lane_ssq.reshape(cores, wout, cout), axis=(0, 1))
    mean = s / cnt
    var = jnp.maximum(sq / cnt - mean * mean, 0.0)
    inv_std = jax.lax.rsqrt(var + _EPS)
    scale_c = gamma.astype(jnp.float32) * inv_std
    shift_c = beta.astype(jnp.float32) - mean * scale_c

    y_t = (y2d[:m].reshape(n, h, wout, cout).transpose(0, 3, 1, 2)
           .reshape(n * cout, h * wout))

    bn = 8
    while n % bn != 0 and bn > 1:
        bn //= 2
    blocks = n // bn
    scale_full = jnp.tile(scale_c, n).reshape(n * cout, 1)
    shift_full = jnp.tile(shift_c, n).reshape(n * cout, 1)

    out2 = pl.pallas_call(
        _bn_relu_kernel,
        out_shape=jax.ShapeDtypeStruct((n * cout, h * wout), jnp.float32),
        grid=(blocks,),
        in_specs=[pl.BlockSpec((bn * cout, h * wout), lambda i: (i, 0)),
                  pl.BlockSpec((bn * cout, 1), lambda i: (i, 0)),
                  pl.BlockSpec((bn * cout, 1), lambda i: (i, 0))],
        out_specs=pl.BlockSpec((bn * cout, h * wout), lambda i: (i, 0)),
        compiler_params=pltpu.CompilerParams(
            dimension_semantics=("parallel",)),
        cost_estimate=pl.CostEstimate(
            flops=3 * m * wc_out, transcendentals=0,
            bytes_accessed=2 * m * wc_out + 4 * m * wc_out),
    )(y_t, scale_full, shift_full)

    return out2.reshape(n, cout, h, wout)


# speedup vs baseline: 2.7254x; 2.7254x over previous
import jax
import jax.numpy as jnp
from jax.experimental import pallas as pl
from jax.experimental.pallas import tpu as pltpu

_EPS = 1e-5


def _stats_kernel(x_ref, w_ref, sum_ref, ssq_ref):
    @pl.when(pl.program_id(1) == 0)
    def _():
        sum_ref[...] = jnp.zeros_like(sum_ref)
        ssq_ref[...] = jnp.zeros_like(ssq_ref)

    y = jnp.dot(x_ref[...], w_ref[...], preferred_element_type=jnp.float32)
    sum_ref[0] += jnp.sum(y, axis=0, keepdims=True)
    ssq_ref[0] += jnp.sum(y * y, axis=0, keepdims=True)


def _conv_bn_relu_kernel(x_ref, w_ref, scale_ref, shift_ref, o_ref):
    y = jnp.dot(x_ref[...], w_ref[...], preferred_element_type=jnp.float32)
    o_ref[...] = jnp.maximum(y * scale_ref[...] + shift_ref[...], 0.0)


def _toeplitz(w_oihw, cin, w, kw, wout, cout):
    wk = (jnp.transpose(w_oihw[:, :, 0, :], (2, 1, 0))
          .reshape(kw * cin, cout))
    stride = w * cin
    ext = stride + cin
    col = jnp.concatenate(
        [wk, jnp.zeros((ext - kw * cin, cout), wk.dtype)], axis=0)
    slab = jnp.tile(col[None], (wout, 1, 1)).reshape(wout * ext, cout)
    w3 = slab[:wout * stride].reshape(wout, stride, cout)
    return (jnp.transpose(w3, (1, 0, 2))
            .reshape(stride, wout * cout).astype(jnp.bfloat16))


def kernel(x_nchw, w_oihw, bias, gamma, beta):
    del bias
    n, cin, h, w = x_nchw.shape
    cout, cin_w, kh, kw = w_oihw.shape
    assert kh == 1 and cin_w == cin and w >= kw
    wout = w - kw + 1
    m = n * h
    wc_in = w * cin
    wc_out = wout * cout

    x2d = (jnp.transpose(x_nchw, (0, 2, 3, 1))
           .reshape(m, wc_in).astype(jnp.bfloat16))
    w_toe = _toeplitz(w_oihw, cin, w, kw, wout, cout)

    tm = min(1024, m)
    tm = max(8, (tm // 8) * 8)
    m_pad = pl.cdiv(m, tm) * tm
    if m_pad != m:
        x2d = jnp.pad(x2d, ((0, m_pad - m), (0, 0)))
    n_tiles = m_pad // tm
    if n_tiles % 2 == 0:
        cores, tiles_per_core = 2, n_tiles // 2
    else:
        cores, tiles_per_core = 1, n_tiles

    lane_sum, lane_ssq = pl.pallas_call(
        _stats_kernel,
        out_shape=(jax.ShapeDtypeStruct((cores, 1, wc_out), jnp.float32),
                   jax.ShapeDtypeStruct((cores, 1, wc_out), jnp.float32)),
        grid=(cores, tiles_per_core),
        in_specs=[pl.BlockSpec((tm, wc_in), lambda c, i, t=tiles_per_core: (c * t + i, 0)),
                  pl.BlockSpec((wc_in, wc_out), lambda c, i: (0, 0))],
        out_specs=(pl.BlockSpec((1, 1, wc_out), lambda c, i: (c, 0, 0)),
                   pl.BlockSpec((1, 1, wc_out), lambda c, i: (c, 0, 0))),
        compiler_params=pltpu.CompilerParams(
            dimension_semantics=("parallel", "arbitrary")),
        cost_estimate=pl.CostEstimate(
            flops=2 * m_pad * wc_in * wc_out, transcendentals=0,
            bytes_accessed=2 * m_pad * wc_in + 2 * wc_in * wc_out),
    )(x2d, w_toe)

    cnt = float(m * wout)
    s = jnp.sum(lane_sum.reshape(cores, wout, cout), axis=(0, 1))
    sq = jnp.sum(lane_ssq.reshape(cores, wout, cout), axis=(0, 1))
    mean = s / cnt
    var = jnp.maximum(sq / cnt - mean * mean, 0.0)
    inv_std = jax.lax.rsqrt(var + _EPS)
    scale_c = gamma.astype(jnp.float32) * inv_std
    shift_c = beta.astype(jnp.float32) - mean * scale_c
    scale_row = jnp.tile(scale_c, wout).reshape(1, wc_out)
    shift_row = jnp.tile(shift_c, wout).reshape(1, wc_out)

    out2d = pl.pallas_call(
        _conv_bn_relu_kernel,
        out_shape=jax.ShapeDtypeStruct((m_pad, wc_out), jnp.float32),
        grid=(n_tiles,),
        in_specs=[pl.BlockSpec((tm, wc_in), lambda i: (i, 0)),
                  pl.BlockSpec((wc_in, wc_out), lambda i: (0, 0)),
                  pl.BlockSpec((1, wc_out), lambda i: (0, 0)),
                  pl.BlockSpec((1, wc_out), lambda i: (0, 0))],
        out_specs=pl.BlockSpec((tm, wc_out), lambda i: (i, 0)),
        compiler_params=pltpu.CompilerParams(
            dimension_semantics=("parallel",)),
        cost_estimate=pl.CostEstimate(
            flops=2 * m_pad * wc_in * wc_out + 3 * m_pad * wc_out,
            transcendentals=0,
            bytes_accessed=(2 * m_pad * wc_in + 2 * wc_in * wc_out
                            + 4 * m_pad * wc_out + 8 * wc_out)),
    )(x2d, w_toe, scale_row, shift_row)

    out = out2d[:m].reshape(n, h, wout, cout)
    return jnp.transpose(out, (0, 3, 1, 2))
